# baseline (speedup 1.0000x reference)
import functools

import jax
import jax.numpy as jnp
from jax import lax
from jax.experimental import pallas as pl
from jax.experimental.pallas import tpu as pltpu
from jax.experimental.pallas import tpu_sc as plsc

NC = 2
NS = 16
L = 16
NW = NC * NS

N = 10000
E = 320000
D = 128
PER_W = E // NW
CHUNK = 80
N_ITERS = PER_W // CHUNK
N_SLOTS = 4
OUT_W = 25 * CHUNK
NRM_W = 640
NRM_STEP = 624


def _sc_body(z_hbm, src_hbm, dst_hbm, out_hbm, sidx_v, didx_v,
             rows, out_v, n_v, nsh, sem_i0, sem_i1, sem_s, sem_d):
    cid = lax.axis_index("c")
    sid = lax.axis_index("s")
    wid = sid * NC + cid
    lane = lax.iota(jnp.int32, L)
    base_w = wid * PER_W

    ci0 = pltpu.async_copy(src_hbm.at[pl.ds(base_w, PER_W)], sidx_v, sem_i0)
    ci1 = pltpu.async_copy(dst_hbm.at[pl.ds(base_w, PER_W)], didx_v, sem_i1)

    nstart = sid * NRM_STEP

    def nrm_chunk(c):
        b = c % N_SLOTS
        row0 = nstart + c * CHUNK
        pltpu.sync_copy(z_hbm.at[pl.ds(row0, CHUNK)], rows[b])

        def ngroup(g, carry):
            eids = g * L + lane

            def nd_blk(j, accs):
                a0, a1 = accs
                for u in range(8):
                    dv = (lane + (j * 16 + u)) & (D - 1)
                    v = plsc.load_gather(rows[b], [eids, dv])
                    a0 = a0 + v * v
                for u in range(8, 16):
                    dv = (lane + (j * 16 + u)) & (D - 1)
                    v = plsc.load_gather(rows[b], [eids, dv])
                    a1 = a1 + v * v
                return a0, a1

            z2 = jnp.zeros((L,), jnp.float32)
            a0, a1 = lax.fori_loop(0, D // 16, nd_blk, (z2, z2))
            n_v[pl.ds(row0 + g * L, L)] = a0 + a1
            return carry

        lax.fori_loop(0, CHUNK // L, ngroup, 0)

    for c in range(NRM_W // CHUNK):
        nrm_chunk(c)

    pltpu.sync_copy(n_v.at[pl.ds(nstart, NRM_W)], nsh.at[pl.ds(nstart, NRM_W)])
    plsc.subcore_barrier()
    pltpu.sync_copy(nsh, n_v)

    ci0.wait()
    ci1.wait()

    def issue_g1(b, chunk):
        off = chunk * CHUNK
        pltpu.async_copy(
            z_hbm.at[sidx_v.at[pl.ds(off, CHUNK)]], rows[b], sem_s[b])

    def wait_g1(b):
        pltpu.make_async_copy(
            z_hbm.at[sidx_v.at[pl.ds(0, CHUNK)]], rows[b], sem_s[b]).wait()

    def issue_g2(b, chunk):
        off = chunk * CHUNK
        pltpu.async_copy(
            z_hbm.at[didx_v.at[pl.ds(off, CHUNK)]], rows[b], sem_d[b],
            add=True)

    def wait_g2(b):
        pltpu.make_async_copy(
            z_hbm.at[didx_v.at[pl.ds(0, CHUNK)]], rows[b], sem_d[b]).wait()

    for b in range(N_SLOTS):
        issue_g1(b, b)
    for b in range(2):
        wait_g1(b)
        issue_g2(b, b)

    def compute(b, chunk):
        cbase = (chunk % 25) * CHUNK

        def group_body(g, carry):
            eids = g * L + lane

            def d_blk(j, accs):
                a0, a1 = accs
                for u in range(8):
                    dv = (lane + (j * 16 + u)) & (D - 1)
                    v = plsc.load_gather(rows[b], [eids, dv])
                    a0 = a0 + v * v
                for u in range(8, 16):
                    dv = (lane + (j * 16 + u)) & (D - 1)
                    v = plsc.load_gather(rows[b], [eids, dv])
                    a1 = a1 + v * v
                return a0, a1

            z2 = jnp.zeros((L,), jnp.float32)
            a0, a1 = lax.fori_loop(0, D // 16, d_blk, (z2, z2))
            sq = a0 + a1
            ns = plsc.load_gather(
                n_v, [sidx_v[pl.ds(chunk * CHUNK + g * L, L)]])
            nt = plsc.load_gather(
                n_v, [didx_v[pl.ds(chunk * CHUNK + g * L, L)]])
            val = 0.5 * (sq - ns - nt)
            out_v[pl.ds(cbase + g * L, L)] = 1.0 / (1.0 + jnp.exp(-val))
            return carry

        lax.fori_loop(0, CHUNK // L, group_body, 0)

    def step(b, chunk):
        wait_g2(b)
        compute(b, chunk)

        @pl.when(chunk + N_SLOTS < N_ITERS)
        def _g1_next():
            issue_g1(b, chunk + N_SLOTS)

        b2 = (b + 2) % N_SLOTS

        @pl.when(chunk + 2 < N_ITERS)
        def _g2_next():
            wait_g1(b2)
            issue_g2(b2, chunk + 2)

        @pl.when(chunk % 25 == 24)
        def _flush():
            pltpu.sync_copy(
                out_v,
                out_hbm.at[pl.ds(base_w + (chunk // 25) * OUT_W, OUT_W)])

    def outer(o, carry):
        for b in range(N_SLOTS):
            step(b, o * N_SLOTS + b)
        return carry

    lax.fori_loop(0, (N_ITERS - 1) // N_SLOTS, outer, 0)
    step(0, N_ITERS - 1)


@jax.jit
def _run(z, src, dst):
    mesh = plsc.VectorSubcoreMesh(
        core_axis_name="c", subcore_axis_name="s",
        num_cores=NC, num_subcores=NS)
    kfn = pl.kernel(
        _sc_body,
        out_type=jax.ShapeDtypeStruct((E,), jnp.float32),
        mesh=mesh,
        scratch_types=[
            pltpu.VMEM((PER_W,), jnp.int32),
            pltpu.VMEM((PER_W,), jnp.int32),
            [pltpu.VMEM((CHUNK, D), jnp.float32) for _ in range(N_SLOTS)],
            pltpu.VMEM((OUT_W,), jnp.float32),
            pltpu.VMEM((N,), jnp.float32),
            pltpu.VMEM_SHARED((N,), jnp.float32),
            pltpu.SemaphoreType.DMA,
            pltpu.SemaphoreType.DMA,
            [pltpu.SemaphoreType.DMA for _ in range(N_SLOTS)],
            [pltpu.SemaphoreType.DMA for _ in range(N_SLOTS)],
        ],
        compiler_params=pltpu.CompilerParams(needs_layout_passes=False),
    )
    return kfn(z, src, dst)


def kernel(z, edge_index):
    src = edge_index[0].astype(jnp.int32)
    dst = edge_index[1].astype(jnp.int32)
    return _run(z, src, dst)


# speedup vs baseline: 1.1704x; 1.1704x over previous
import functools

import jax
import jax.numpy as jnp
from jax import lax
from jax.experimental import pallas as pl
from jax.experimental.pallas import tpu as pltpu
from jax.experimental.pallas import tpu_sc as plsc

NC = 2
NS = 16
L = 16
NW = NC * NS

N = 10000
E = 320000
D = 128
PER_W = E // NW
CHUNK = 80
N_ITERS = PER_W // CHUNK
RS = 3
KS = 4
UNROLL = 12
N_MAIN = 120
OUT_W = 25 * CHUNK
ZS_STEP = 624
NRM_W = 640
NRM_STEP = 624


def _sc_body(z_hbm, src_hbm, dst_hbm, out_hbm,
             rows, is_v, id_v, out_v, n_v, zsh, nsh,
             sem_g1, sem_g2, sem_ix):
    cid = lax.axis_index("c")
    sid = lax.axis_index("s")
    lane = lax.iota(jnp.int32, L)
    base_w = (sid * NC + cid) * PER_W

    zrow = sid * ZS_STEP
    pltpu.sync_copy(z_hbm.at[pl.ds(zrow, ZS_STEP)],
                    zsh.at[pl.ds(zrow, ZS_STEP)])

    @pl.when(sid == NS - 1)
    def _stage_tail():
        pltpu.sync_copy(z_hbm.at[pl.ds(NS * ZS_STEP, N - NS * ZS_STEP)],
                        zsh.at[pl.ds(NS * ZS_STEP, N - NS * ZS_STEP)])

    plsc.subcore_barrier()

    nstart = sid * NRM_STEP

    def sumsq_groups(buf, out_off, n_groups):
        def ngroup(g, carry):
            eids = g * L + lane

            def nd_blk(j, accs):
                a0, a1 = accs
                for u in range(8):
                    dv = (lane + (j * 16 + u)) & (D - 1)
                    v = plsc.load_gather(buf, [eids, dv])
                    a0 = a0 + v * v
                for u in range(8, 16):
                    dv = (lane + (j * 16 + u)) & (D - 1)
                    v = plsc.load_gather(buf, [eids, dv])
                    a1 = a1 + v * v
                return a0, a1

            z2 = jnp.zeros((L,), jnp.float32)
            a0, a1 = lax.fori_loop(0, D // 16, nd_blk, (z2, z2))
            n_v[pl.ds(out_off + g * L, L)] = a0 + a1
            return carry

        lax.fori_loop(0, n_groups, ngroup, 0)

    for c in range(NRM_W // CHUNK):
        row0 = nstart + c * CHUNK
        pltpu.sync_copy(zsh.at[pl.ds(row0, CHUNK)], rows[c % RS])
        sumsq_groups(rows[c % RS], row0, CHUNK // L)

    pltpu.sync_copy(n_v.at[pl.ds(nstart, NRM_W)],
                    nsh.at[pl.ds(nstart, NRM_W)])
    plsc.subcore_barrier()
    pltpu.sync_copy(nsh, n_v)

    def issue_idx(k, chunk):
        off = base_w + chunk * CHUNK
        pltpu.async_copy(src_hbm.at[pl.ds(off, CHUNK)], is_v[k], sem_ix[k])
        pltpu.async_copy(dst_hbm.at[pl.ds(off, CHUNK)], id_v[k], sem_ix[k])

    def wait_idx(k):
        pltpu.make_async_copy(src_hbm.at[pl.ds(0, CHUNK)], is_v[k],
                              sem_ix[k]).wait()
        pltpu.make_async_copy(dst_hbm.at[pl.ds(0, CHUNK)], id_v[k],
                              sem_ix[k]).wait()

    def issue_g1(b, k):
        pltpu.async_copy(zsh.at[is_v[k]], rows[b], sem_g1[b])

    def wait_g1(b):
        pltpu.make_async_copy(zsh.at[is_v[0]], rows[b], sem_g1[b]).wait()

    def issue_g2(b, k):
        pltpu.async_copy(zsh.at[id_v[k]], rows[b], sem_g2[b], add=True)

    def wait_g2(b):
        pltpu.make_async_copy(zsh.at[id_v[0]], rows[b], sem_g2[b]).wait()

    def compute(b, k, chunk):
        cbase = (chunk % 25) * CHUNK

        def group_body(g, carry):
            eids = g * L + lane

            def d_blk(j, accs):
                a0, a1 = accs
                for u in range(8):
                    dv = (lane + (j * 16 + u)) & (D - 1)
                    v = plsc.load_gather(rows[b], [eids, dv])
                    a0 = a0 + v * v
                for u in range(8, 16):
                    dv = (lane + (j * 16 + u)) & (D - 1)
                    v = plsc.load_gather(rows[b], [eids, dv])
                    a1 = a1 + v * v
                return a0, a1

            z2 = jnp.zeros((L,), jnp.float32)
            a0, a1 = lax.fori_loop(0, D // 16, d_blk, (z2, z2))
            sq = a0 + a1
            ns = plsc.load_gather(n_v, [is_v[k][pl.ds(g * L, L)]])
            nt = plsc.load_gather(n_v, [id_v[k][pl.ds(g * L, L)]])
            val = 0.5 * (sq - ns - nt)
            out_v[pl.ds(cbase + g * L, L)] = 1.0 / (1.0 + jnp.exp(-val))
            return carry

        lax.fori_loop(0, CHUNK // L, group_body, 0)

    def step_main(chunk, j):
        b, k = j % RS, j % KS
        wait_g2(b)
        compute(b, k, chunk)

        @pl.when(chunk + 4 < N_ITERS)
        def _ix():
            issue_idx((j + 4) % KS, chunk + 4)

        @pl.when(chunk + 3 < N_ITERS)
        def _g1():
            wait_idx((j + 3) % KS)
            issue_g1((j + 3) % RS, (j + 3) % KS)

        @pl.when(chunk + 2 < N_ITERS)
        def _g2():
            wait_g1((j + 2) % RS)
            issue_g2((j + 2) % RS, (j + 2) % KS)

        @pl.when(chunk % 25 == 24)
        def _flush():
            pltpu.sync_copy(
                out_v,
                out_hbm.at[pl.ds(base_w + (chunk // 25) * OUT_W, OUT_W)])

    def step_tail(c):
        j = c % UNROLL
        b, k = j % RS, j % KS
        wait_g2(b)
        compute(b, k, c)
        if c + 4 < N_ITERS:
            issue_idx((j + 4) % KS, c + 4)
        if c + 3 < N_ITERS:
            wait_idx((j + 3) % KS)
            issue_g1((j + 3) % RS, (j + 3) % KS)
        if c + 2 < N_ITERS:
            wait_g1((j + 2) % RS)
            issue_g2((j + 2) % RS, (j + 2) % KS)
        if c % 25 == 24:
            pltpu.sync_copy(
                out_v,
                out_hbm.at[pl.ds(base_w + (c // 25) * OUT_W, OUT_W)])

    for k in range(KS):
        issue_idx(k, k)
    for b in range(RS):
        wait_idx(b % KS)
        issue_g1(b, b % KS)
    for b in range(2):
        wait_g1(b)
        issue_g2(b, b % KS)

    def outer(o, carry):
        for j in range(UNROLL):
            step_main(o * UNROLL + j, j)
        return carry

    lax.fori_loop(0, N_MAIN // UNROLL, outer, 0)
    for c in range(N_MAIN, N_ITERS):
        step_tail(c)


@jax.jit
def _run(z, src, dst):
    mesh = plsc.VectorSubcoreMesh(
        core_axis_name="c", subcore_axis_name="s",
        num_cores=NC, num_subcores=NS)
    kfn = pl.kernel(
        _sc_body,
        out_type=jax.ShapeDtypeStruct((E,), jnp.float32),
        mesh=mesh,
        scratch_types=[
            [pltpu.VMEM((CHUNK, D), jnp.float32) for _ in range(RS)],
            [pltpu.VMEM((CHUNK,), jnp.int32) for _ in range(KS)],
            [pltpu.VMEM((CHUNK,), jnp.int32) for _ in range(KS)],
            pltpu.VMEM((OUT_W,), jnp.float32),
            pltpu.VMEM((N,), jnp.float32),
            pltpu.VMEM_SHARED((N, D), jnp.float32),
            pltpu.VMEM_SHARED((N,), jnp.float32),
            [pltpu.SemaphoreType.DMA for _ in range(RS)],
            [pltpu.SemaphoreType.DMA for _ in range(RS)],
            [pltpu.SemaphoreType.DMA for _ in range(KS)],
        ],
        compiler_params=pltpu.CompilerParams(needs_layout_passes=False),
    )
    return kfn(z, src, dst)


def kernel(z, edge_index):
    src = edge_index[0].astype(jnp.int32)
    dst = edge_index[1].astype(jnp.int32)
    return _run(z, src, dst)
